# baseline (speedup 1.0000x reference)
import jax
import jax.numpy as jnp
from jax.experimental import pallas as pl
from jax.experimental.pallas import tpu as pltpu

B = 8
MAXN = 1024
HID = 64
OUT = 16
N1 = 103
N2 = 11

_BF = jnp.bfloat16
GPB = 4


def _mm(a, b):
    return jax.lax.dot_general(a, b, (((1,), (0,)), ((), ())),
                               preferred_element_type=jnp.float32)


def _mm_t(a, b):
    return jax.lax.dot_general(a, b, (((0,), (0,)), ((), ())),
                               preferred_element_type=jnp.float32)


def _softmax(z):
    z = z - jnp.max(z, axis=-1, keepdims=True)
    e = jnp.exp(z)
    return e * (1.0 / jnp.sum(e, axis=-1, keepdims=True))


def _diffpool_body(x_ref, adj_ref, Wc_ref, Ws_ref, Wh_ref,
                   W2c_ref, W2s_ref, W2h_ref, W3a_ref, W3b_ref,
                   out_ref):
    G = range(GPB)
    relu = jax.nn.relu

    adj = [adj_ref[g].astype(_BF) for g in G]
    t = [_mm(adj[g], x_ref[g].astype(_BF)) for g in G]
    sh = [relu(_mm(t[g], Wc_ref[...])).astype(_BF) for g in G]
    uv = [_mm(adj[g], sh[g]) for g in G]
    s = [relu(_mm(uv[g], Ws_ref[...])) for g in G]
    h = [relu(_mm(uv[g], Wh_ref[...])).astype(_BF) for g in G]

    ss = [_softmax(s[g]).astype(_BF) for g in G]
    x_p = [_mm_t(ss[g], h[g]) for g in G]
    w = [_mm(adj[g], ss[g]).astype(_BF) for g in G]
    a_p = [_mm_t(ss[g], w[g]) for g in G]

    t2 = [_mm(a_p[g], x_p[g]) for g in G]
    sh2 = [relu(_mm(t2[g], W2c_ref[...])) for g in G]
    uv2 = [_mm(a_p[g], sh2[g]) for g in G]
    s2 = [relu(_mm(uv2[g], W2s_ref[...])) for g in G]
    h2 = [relu(_mm(uv2[g], W2h_ref[...])) for g in G]

    ss2 = [_softmax(s2[g]) for g in G]
    x_q = [_mm_t(ss2[g], h2[g]) for g in G]
    w2 = [_mm(a_p[g], ss2[g]) for g in G]
    a_q = [_mm_t(ss2[g], w2[g]) for g in G]

    z1 = [relu(_mm(_mm(a_q[g], x_q[g]), W3a_ref[...])) for g in G]
    z2 = [relu(_mm(_mm(a_q[g], z1[g]), W3b_ref[...])) for g in G]
    for g in G:
        out_ref[g, 0] = jnp.mean(z2[g], axis=0)


def kernel(x, adj, W1p0, W1p1, W1e0, W1e1, W2p0, W2p1, W2e0, W2e1, W3a, W3b):
    P = 128
    Wc = jnp.concatenate(
        [W1p0, jnp.zeros((HID, P - N1), jnp.float32), W1e0], axis=1)
    Ws = jnp.concatenate(
        [W1p1, jnp.zeros((P - N1 + HID, N1), jnp.float32)], axis=0)
    Wh = jnp.concatenate(
        [jnp.zeros((P, HID), jnp.float32), W1e1], axis=0)
    P2 = 16
    W2c = jnp.concatenate(
        [W2p0, jnp.zeros((HID, P2 - N2), jnp.float32), W2e0], axis=1)
    W2s = jnp.concatenate(
        [W2p1, jnp.zeros((P2 - N2 + HID, N2), jnp.float32)], axis=0)
    W2h = jnp.concatenate(
        [jnp.zeros((P2, HID), jnp.float32), W2e1], axis=0)

    w_spec = lambda shp: pl.BlockSpec(shp, lambda b: (0,) * len(shp))
    out = pl.pallas_call(
        _diffpool_body,
        grid=(B // GPB,),
        in_specs=[
            pl.BlockSpec((GPB, MAXN, HID), lambda b: (b, 0, 0)),
            pl.BlockSpec((GPB, MAXN, MAXN), lambda b: (b, 0, 0)),
            w_spec(Wc.shape), w_spec(Ws.shape), w_spec(Wh.shape),
            w_spec(W2c.shape), w_spec(W2s.shape), w_spec(W2h.shape),
            w_spec(W3a.shape), w_spec(W3b.shape),
        ],
        out_specs=pl.BlockSpec((GPB, 1, OUT), lambda b: (b, 0, 0)),
        out_shape=jax.ShapeDtypeStruct((B, 1, OUT), jnp.float32),
        compiler_params=pltpu.CompilerParams(
            dimension_semantics=("parallel",),
        ),
    )(x, adj, Wc, Ws, Wh, W2c, W2s, W2h, W3a, W3b)
    return out.reshape(B, OUT)
